# baseline (speedup 1.0000x reference)
import jax
import jax.numpy as jnp
from jax import lax
from jax.experimental import pallas as pl
from jax.experimental.pallas import tpu as pltpu
from jax.experimental.pallas import tpu_sc as plsc

_B = 16384
_L = 200
_VOCAB = 1000000
_DIM = 32

_NC = 2
_NS = 16
_LANES = 16
_NW = _NC * _NS
_GRP = _B // _LANES
_GPW = _GRP // _NW
_TOK = _L * _LANES
_CH = 128
_NCH = _TOK // _CH

_FOLD = 8
_WIDE = _DIM * _FOLD
_VROWS = _VOCAB // _FOLD
_VB = 1000


def _proj_body(s_ref, tbl_ref, out_ref):
    out_ref[...] = jnp.dot(tbl_ref[...], s_ref[...],
                           preferred_element_type=jnp.float32)


def _project(embd_wide, sel):
    return pl.pallas_call(
        _proj_body,
        grid=(_VROWS // _VB,),
        in_specs=[
            pl.BlockSpec((_WIDE, _FOLD), lambda i: (0, 0)),
            pl.BlockSpec((_VB, _WIDE), lambda i: (i, 0)),
        ],
        out_specs=pl.BlockSpec((_VB, _FOLD), lambda i: (i, 0)),
        out_shape=jax.ShapeDtypeStruct((_VROWS, _FOLD), jnp.float32),
    )(sel, embd_wide)


def _pool_body(p_hbm, ids_hbm, att_hbm, bias_hbm, tidx_hbm, out_hbm,
               tidx_v, gidx_v, idxt_v, attt_v, val_v,
               bias_v, out_v, ish, ash, semt, semg):
    cid = lax.axis_index("c")
    sid = lax.axis_index("s")
    wid = sid * _NC + cid
    pltpu.sync_copy(bias_hbm, bias_v)
    pltpu.sync_copy(tidx_hbm, tidx_v)
    soff = (sid * _TOK).astype(jnp.int32)
    for k in range(_L):
        ds = pl.ds(k * _LANES, _LANES)
        gidx_v[ds] = tidx_v[ds] + soff

    def group_body(gl, carry):
        g = wid * _GPW + gl
        my_ish = ish.at[pl.ds(soff, _TOK)]
        my_ash = ash.at[pl.ds(soff, _TOK)]
        pltpu.sync_copy(ids_hbm.at[pl.ds(g * _TOK, _TOK)], my_ish)
        pltpu.sync_copy(att_hbm.at[pl.ds(g * _TOK, _TOK)], my_ash)
        tcopies = []
        for j in range(_NCH):
            ds = pl.ds(j * _CH, _CH)
            tcopies.append(pltpu.async_copy(
                ish.at[gidx_v.at[ds]], idxt_v.at[ds], semt))
            tcopies.append(pltpu.async_copy(
                ash.at[gidx_v.at[ds]], attt_v.at[ds], semt))
        for c in tcopies:
            c.wait()
        gcopies = [
            pltpu.async_copy(
                p_hbm.at[idxt_v.at[pl.ds(j * _CH, _CH)]],
                val_v.at[pl.ds(j * _CH, _CH)],
                semg,
            )
            for j in range(_NCH)
        ]
        for c in gcopies:
            c.wait()

        def tok_body(i, tc):
            acc, den = tc
            a = attt_v[pl.ds(i * _LANES, _LANES)]
            v = val_v[pl.ds(i * _LANES, _LANES)]
            return acc + v * a, den + a

        zero = jnp.zeros((_LANES,), jnp.float32)
        acc, den = lax.fori_loop(0, _L, tok_body, (zero, zero))
        logit = acc / den + bias_v[...]
        out_v[...] = 1.0 / (1.0 + jnp.exp(-logit))
        pltpu.sync_copy(out_v, out_hbm.at[pl.ds(g * _LANES, _LANES)])
        return carry

    lax.fori_loop(0, _GPW, group_body, 0)


def _pool(p, ids_g, att_g, bias16, tidx):
    mesh = plsc.VectorSubcoreMesh(
        core_axis_name="c", subcore_axis_name="s",
        num_cores=_NC, num_subcores=_NS,
    )
    return pl.kernel(
        _pool_body,
        out_type=jax.ShapeDtypeStruct((_B,), jnp.float32),
        mesh=mesh,
        scratch_types=[
            pltpu.VMEM((_TOK,), jnp.int32),
            pltpu.VMEM((_TOK,), jnp.int32),
            pltpu.VMEM((_TOK,), jnp.int32),
            pltpu.VMEM((_TOK,), jnp.float32),
            pltpu.VMEM((_TOK,), jnp.float32),
            pltpu.VMEM((_LANES,), jnp.float32),
            pltpu.VMEM((_LANES,), jnp.float32),
            pltpu.VMEM_SHARED((_NS * _TOK,), jnp.int32),
            pltpu.VMEM_SHARED((_NS * _TOK,), jnp.float32),
            pltpu.SemaphoreType.DMA,
            pltpu.SemaphoreType.DMA,
        ],
    )(p, ids_g, att_g, bias16, tidx)


def kernel(ids, att_ids, embd_weight, linear_weight, linear_bias):
    ids_g = ids.astype(jnp.int32).reshape(-1)
    att_g = att_ids.astype(jnp.float32).reshape(-1)
    bias16 = jnp.broadcast_to(linear_bias.astype(jnp.float32), (_LANES,))
    embd_wide = embd_weight.reshape(_VROWS, _WIDE)
    w0 = linear_weight.astype(jnp.float32).reshape(_DIM)
    sel = jnp.kron(jnp.eye(_FOLD, dtype=jnp.float32), w0[:, None])
    t = jnp.arange(_TOK, dtype=jnp.int32)
    tidx = (t % _LANES) * _L + t // _LANES
    p = _project(embd_wide, sel).reshape(_VOCAB)
    return _pool(p, ids_g, att_g, bias16, tidx)


# speedup vs baseline: 1.0516x; 1.0516x over previous
import jax
import jax.numpy as jnp
from jax import lax
from jax.experimental import pallas as pl
from jax.experimental.pallas import tpu as pltpu
from jax.experimental.pallas import tpu_sc as plsc

_B = 16384
_L = 200
_VOCAB = 1000000
_DIM = 32

_NC = 2
_NS = 16
_LANES = 16
_NW = _NC * _NS
_GRP = _B // _LANES
_GPW = _GRP // _NW
_TOK = _L * _LANES
_CH = 128
_NCH = _TOK // _CH

_FOLD = 8
_WIDE = _DIM * _FOLD
_VROWS = _VOCAB // _FOLD
_VB = 1024
_PGRID = -(-_VROWS // _VB)


def _proj_body(s_ref, tbl_ref, out_ref):
    out_ref[...] = lax.dot_general(
        s_ref[...], tbl_ref[...], (((0,), (1,)), ((), ())),
        preferred_element_type=jnp.float32)


def _project(embd_wide, sel):
    return pl.pallas_call(
        _proj_body,
        grid=(_PGRID,),
        in_specs=[
            pl.BlockSpec((_WIDE, _FOLD), lambda i: (0, 0)),
            pl.BlockSpec((_VB, _WIDE), lambda i: (i, 0)),
        ],
        out_specs=pl.BlockSpec((_FOLD, _VB), lambda i: (0, i)),
        out_shape=jax.ShapeDtypeStruct((_FOLD, _VROWS), jnp.float32),
    )(sel, embd_wide)


def _pool_body(p_hbm, ids_hbm, att_hbm, bias_hbm, tidx_hbm, out_hbm,
               tidx_v, gidx_v, idxt_v, attt_v, val_v,
               bias_v, out_v, ish, ash, semt, semg):
    cid = lax.axis_index("c")
    sid = lax.axis_index("s")
    wid = sid * _NC + cid
    pltpu.sync_copy(bias_hbm, bias_v)
    pltpu.sync_copy(tidx_hbm, tidx_v)
    soff = (sid * _TOK).astype(jnp.int32)
    for k in range(_L):
        ds = pl.ds(k * _LANES, _LANES)
        gidx_v[ds] = tidx_v[ds] + soff

    def group_body(gl, carry):
        g = wid * _GPW + gl
        my_ish = ish.at[pl.ds(soff, _TOK)]
        my_ash = ash.at[pl.ds(soff, _TOK)]
        pltpu.sync_copy(ids_hbm.at[pl.ds(g * _TOK, _TOK)], my_ish)
        pltpu.sync_copy(att_hbm.at[pl.ds(g * _TOK, _TOK)], my_ash)
        tcopies = []
        for j in range(_NCH):
            ds = pl.ds(j * _CH, _CH)
            tcopies.append(pltpu.async_copy(
                ish.at[gidx_v.at[ds]], idxt_v.at[ds], semt))
            tcopies.append(pltpu.async_copy(
                ash.at[gidx_v.at[ds]], attt_v.at[ds], semt))
        for c in tcopies:
            c.wait()
        gcopies = [
            pltpu.async_copy(
                p_hbm.at[idxt_v.at[pl.ds(j * _CH, _CH)]],
                val_v.at[pl.ds(j * _CH, _CH)],
                semg,
            )
            for j in range(_NCH)
        ]
        for c in gcopies:
            c.wait()

        def tok_body(i, tc):
            acc, den = tc
            a = attt_v[pl.ds(i * _LANES, _LANES)]
            v = val_v[pl.ds(i * _LANES, _LANES)]
            return acc + v * a, den + a

        zero = jnp.zeros((_LANES,), jnp.float32)
        acc, den = lax.fori_loop(0, _L, tok_body, (zero, zero))
        logit = acc / den + bias_v[...]
        out_v[...] = 1.0 / (1.0 + jnp.exp(-logit))
        pltpu.sync_copy(out_v, out_hbm.at[pl.ds(g * _LANES, _LANES)])
        return carry

    lax.fori_loop(0, _GPW, group_body, 0)


def _pool(p, ids_g, att_g, bias16, tidx):
    mesh = plsc.VectorSubcoreMesh(
        core_axis_name="c", subcore_axis_name="s",
        num_cores=_NC, num_subcores=_NS,
    )
    return pl.kernel(
        _pool_body,
        out_type=jax.ShapeDtypeStruct((_B,), jnp.float32),
        mesh=mesh,
        scratch_types=[
            pltpu.VMEM((_TOK,), jnp.int32),
            pltpu.VMEM((_TOK,), jnp.int32),
            pltpu.VMEM((_TOK,), jnp.int32),
            pltpu.VMEM((_TOK,), jnp.float32),
            pltpu.VMEM((_TOK,), jnp.float32),
            pltpu.VMEM((_LANES,), jnp.float32),
            pltpu.VMEM((_LANES,), jnp.float32),
            pltpu.VMEM_SHARED((_NS * _TOK,), jnp.int32),
            pltpu.VMEM_SHARED((_NS * _TOK,), jnp.float32),
            pltpu.SemaphoreType.DMA,
            pltpu.SemaphoreType.DMA,
        ],
    )(p, ids_g, att_g, bias16, tidx)


def kernel(ids, att_ids, embd_weight, linear_weight, linear_bias):
    ids32 = ids.astype(jnp.int32)
    ids_g = ((ids32 & 7) * _VROWS + (ids32 >> 3)).reshape(-1)
    att_g = att_ids.astype(jnp.float32).reshape(-1)
    bias16 = jnp.broadcast_to(linear_bias.astype(jnp.float32), (_LANES,))
    embd_wide = embd_weight.reshape(_VROWS, _WIDE)
    w0 = linear_weight.astype(jnp.float32).reshape(_DIM)
    sel = jnp.kron(jnp.eye(_FOLD, dtype=jnp.float32), w0[:, None])
    t = jnp.arange(_TOK, dtype=jnp.int32)
    tidx = (t % _LANES) * _L + t // _LANES
    p = _project(embd_wide, sel).reshape(_VOCAB)
    return _pool(p, ids_g, att_g, bias16, tidx)
